# baseline (speedup 1.0000x reference)
import functools

import jax
import jax.numpy as jnp
from jax import lax
from jax.experimental import pallas as pl
from jax.experimental.pallas import tpu as pltpu
from jax.experimental.pallas import tpu_sc as plsc

B = 1024
S = 200
C = 4
H = 64
HALF = 1
BH = B // HALF
T = BH * S
N = T * C
NW = 32
ROWS_PER_W = N // NW
CHUNK = 128
NCHUNK = ROWS_PER_W // CHUNK

_sc_mesh = plsc.VectorSubcoreMesh(core_axis_name="c", subcore_axis_name="s")

V = 1000000
_VB = 2048
_TR_GRID = 245
VSPLIT = _VB * _TR_GRID


def _tr_body(wa_ref, wb_ref, out_ref):
    a = lax.transpose(wa_ref[...], (1, 0))
    b = lax.transpose(wb_ref[...], (1, 0))
    out_ref[...] = jnp.concatenate([a, b], axis=1)


def _tc_detranspose(wt):
    return pl.pallas_call(
        _tr_body,
        grid=(_TR_GRID,),
        in_specs=[
            pl.BlockSpec((H, _VB), lambda i: (0, i)),
            pl.BlockSpec((H, _VB),
                         lambda i: (0, jnp.minimum(_TR_GRID + i, V // _VB))),
        ],
        out_specs=pl.BlockSpec((_VB, 128), lambda i: (i, 0)),
        out_shape=jax.ShapeDtypeStruct((VSPLIT, 128), jnp.float32),
    )(wt, wt)


@functools.partial(
    pl.kernel,
    out_type=jax.ShapeDtypeStruct((C, T, H), jnp.float32),
    mesh=_sc_mesh,
    scratch_types=[
        pltpu.VMEM((NCHUNK, CHUNK), jnp.int32),
        pltpu.VMEM((CHUNK, H), jnp.float32),
        pltpu.VMEM((CHUNK, H), jnp.float32),
        pltpu.SemaphoreType.DMA,
        pltpu.SemaphoreType.DMA,
    ],
    compiler_params=pltpu.CompilerParams(use_tc_tiling_on_sc=False),
)
def _sc_gather(x_hbm, table_hbm, out_hbm, idx_v, rows0, rows1, sem0, sem1):
    wid = lax.axis_index("s") * 2 + lax.axis_index("c")
    ch = wid // 8
    base = (wid % 8) * ROWS_PER_W
    pltpu.sync_copy(x_hbm.at[wid], idx_v)
    pltpu.async_copy(table_hbm.at[idx_v.at[0]], rows0, sem0)

    @pl.loop(0, NCHUNK, step=2)
    def _chunk(j):
        pltpu.make_async_copy(table_hbm.at[idx_v.at[j]], rows0, sem0).wait()
        pltpu.async_copy(table_hbm.at[idx_v.at[j + 1]], rows1, sem1)
        pltpu.sync_copy(rows0, out_hbm.at[ch, pl.ds(base + j * CHUNK, CHUNK)])
        pltpu.make_async_copy(
            table_hbm.at[idx_v.at[j + 1]], rows1, sem1).wait()

        @pl.when(j + 2 < NCHUNK)
        def _():
            pltpu.async_copy(table_hbm.at[idx_v.at[j + 2]], rows0, sem0)

        pltpu.sync_copy(
            rows1, out_hbm.at[ch, pl.ds(base + (j + 1) * CHUNK, CHUNK)])


TP = T // 2
_PAIR_BLK = 800
_GRID = TP // _PAIR_BLK


def _tc_body(g_ref, pce_ref, fw_ref, lng_ref, fcw_ref, fcb2_ref, flng2_ref,
             flnb2_ref, out_ref):
    f32 = jnp.float32
    r = lax.broadcasted_iota(jnp.int32, (128, 128), 0)
    c2 = lax.broadcasted_iota(jnp.int32, (128, 128), 1)
    mavg = jnp.where((r < 64) == (c2 < 64), 1.0 / 64, 0.0).astype(f32)

    fw = fw_ref[...]
    e = jnp.exp(fw - jnp.max(fw))
    w = e / jnp.sum(e)

    def ln_stats(y):
        m = lax.dot_general(y, mavg, (((1,), (0,)), ((), ())),
                            preferred_element_type=f32)
        d = y - m
        v = lax.dot_general(d * d, mavg, (((1,), (0,)), ((), ())),
                            preferred_element_type=f32)
        return d * lax.rsqrt(v + 1e-5)

    t = None
    for c in range(C):
        z = ln_stats(g_ref[c] + pce_ref[c])
        zc = z * w[0, c]
        t = zc if t is None else t + zc
    zg = t * lng_ref[...]

    fcw = fcw_ref[...]
    ha = lax.dot_general(zg[:, :H], fcw, (((1,), (1,)), ((), ())),
                         preferred_element_type=f32)
    hb = lax.dot_general(zg[:, H:], fcw, (((1,), (1,)), ((), ())),
                         preferred_element_type=f32)
    h = jnp.concatenate([ha, hb], axis=1) + fcb2_ref[...]
    out = ln_stats(h)
    out_ref[...] = out * flng2_ref[...] + flnb2_ref[...]


def _tc_fuse(g2, pce, fuse_w, lng2, fc_W, fcb2, flng2, flnb2):
    const = lambda shape: pl.BlockSpec(shape, lambda i: (0,) * len(shape))
    return pl.pallas_call(
        _tc_body,
        grid=(_GRID,),
        in_specs=[
            pl.BlockSpec((C, _PAIR_BLK, 128), lambda i: (0, i, 0)),
            const((C, _PAIR_BLK, 128)),
            const((1, C)),
            const((1, 128)),
            const((H, H)),
            const((1, 128)),
            const((1, 128)),
            const((1, 128)),
        ],
        out_specs=pl.BlockSpec((_PAIR_BLK, 128), lambda i: (i, 0)),
        out_shape=jax.ShapeDtypeStruct((TP, 128), jnp.float32),
    )(g2, pce, fuse_w, lng2, fc_W, fcb2, flng2, flnb2)


def kernel(x, masks, word_emb, pos_emb, chan_emb, ln_g, ln_b, fuse_w, fc_W,
           fc_b, fln_g, fln_b):
    pce = (pos_emb[None, :, :] + chan_emb[:, None, :]).reshape(C, S // 2, 128)
    pce = jnp.tile(pce, (1, _PAIR_BLK // (S // 2), 1))
    pair = lambda v: jnp.concatenate([v, v]).reshape(1, 128)
    fcb2 = pair(ln_b @ fc_W.T + fc_b)

    w2 = _tc_detranspose(word_emb.T)
    w2lin = w2.reshape(2 * VSPLIT, H)
    xi = jnp.where(x < VSPLIT, 2 * x, 2 * (x - VSPLIT) + 1)

    outs = []
    for k in range(HALF):
        xk = lax.slice_in_dim(xi, k * BH, (k + 1) * BH, axis=0)
        xt = xk.transpose(2, 0, 1).reshape(NW, NCHUNK, CHUNK)
        g = _sc_gather(xt, w2lin)
        g2 = g.reshape(C, TP, 128)
        outs.append(_tc_fuse(
            g2, pce, fuse_w.reshape(1, C), pair(ln_g), fc_W,
            fcb2, pair(fln_g), pair(fln_b),
        ))
    out = jnp.concatenate(outs, axis=0)
    return (out.reshape(B, S, H), masks)


# speedup vs baseline: 1.0430x; 1.0430x over previous
import functools

import jax
import jax.numpy as jnp
from jax import lax
from jax.experimental import pallas as pl
from jax.experimental.pallas import tpu as pltpu
from jax.experimental.pallas import tpu_sc as plsc

B = 1024
S = 200
C = 4
H = 64
HALF = 2
BH = B // HALF
T = BH * S
N = T * C
NW = 32
ROWS_PER_W = N // NW
CHUNK = 128
NCHUNK = ROWS_PER_W // CHUNK

_sc_mesh = plsc.VectorSubcoreMesh(core_axis_name="c", subcore_axis_name="s")

V = 1000000
_VB = 2048
_TR_GRID = 245
VSPLIT = _VB * _TR_GRID


def _tr_body(wa_ref, wb_ref, out_ref):
    a = lax.transpose(wa_ref[...], (1, 0))
    b = lax.transpose(wb_ref[...], (1, 0))
    out_ref[...] = jnp.concatenate([a, b], axis=1)


def _tc_detranspose(wt):
    return pl.pallas_call(
        _tr_body,
        grid=(_TR_GRID,),
        in_specs=[
            pl.BlockSpec((H, _VB), lambda i: (0, i)),
            pl.BlockSpec((H, _VB),
                         lambda i: (0, jnp.minimum(_TR_GRID + i, V // _VB))),
        ],
        out_specs=pl.BlockSpec((_VB, 128), lambda i: (i, 0)),
        out_shape=jax.ShapeDtypeStruct((VSPLIT, 128), jnp.float32),
    )(wt, wt)


@functools.partial(
    pl.kernel,
    out_type=jax.ShapeDtypeStruct((C, T, H), jnp.float32),
    mesh=_sc_mesh,
    scratch_types=[
        pltpu.VMEM((NCHUNK, CHUNK), jnp.int32),
        pltpu.VMEM((CHUNK, H), jnp.float32),
        pltpu.VMEM((CHUNK, H), jnp.float32),
        pltpu.SemaphoreType.DMA,
        pltpu.SemaphoreType.DMA,
    ],
    compiler_params=pltpu.CompilerParams(use_tc_tiling_on_sc=False),
)
def _sc_gather(x_hbm, table_hbm, out_hbm, idx_v, rows0, rows1, sem0, sem1):
    wid = lax.axis_index("s") * 2 + lax.axis_index("c")
    ch = wid // 8
    base = (wid % 8) * ROWS_PER_W
    pltpu.sync_copy(x_hbm.at[wid], idx_v)
    pltpu.async_copy(table_hbm.at[idx_v.at[0]], rows0, sem0)

    @pl.loop(0, NCHUNK, step=2)
    def _chunk(j):
        pltpu.make_async_copy(table_hbm.at[idx_v.at[j]], rows0, sem0).wait()
        pltpu.async_copy(table_hbm.at[idx_v.at[j + 1]], rows1, sem1)
        pltpu.sync_copy(rows0, out_hbm.at[ch, pl.ds(base + j * CHUNK, CHUNK)])
        pltpu.make_async_copy(
            table_hbm.at[idx_v.at[j + 1]], rows1, sem1).wait()

        @pl.when(j + 2 < NCHUNK)
        def _():
            pltpu.async_copy(table_hbm.at[idx_v.at[j + 2]], rows0, sem0)

        pltpu.sync_copy(
            rows1, out_hbm.at[ch, pl.ds(base + (j + 1) * CHUNK, CHUNK)])


TP = T // 2
_PAIR_BLK = 800
_GRID = TP // _PAIR_BLK


def _tc_body(g_ref, pce_ref, fw_ref, lng_ref, fcw_ref, fcb2_ref, flng2_ref,
             flnb2_ref, out_ref):
    f32 = jnp.float32
    r = lax.broadcasted_iota(jnp.int32, (128, 128), 0)
    c2 = lax.broadcasted_iota(jnp.int32, (128, 128), 1)
    mavg = jnp.where((r < 64) == (c2 < 64), 1.0 / 64, 0.0).astype(f32)

    fw = fw_ref[...]
    e = jnp.exp(fw - jnp.max(fw))
    w = e / jnp.sum(e)

    def ln_stats(y):
        m = lax.dot_general(y, mavg, (((1,), (0,)), ((), ())),
                            preferred_element_type=f32)
        d = y - m
        v = lax.dot_general(d * d, mavg, (((1,), (0,)), ((), ())),
                            preferred_element_type=f32)
        return d * lax.rsqrt(v + 1e-5)

    t = None
    for c in range(C):
        z = ln_stats(g_ref[c] + pce_ref[c])
        zc = z * w[0, c]
        t = zc if t is None else t + zc
    zg = t * lng_ref[...]

    fcw = fcw_ref[...]
    ha = lax.dot_general(zg[:, :H], fcw, (((1,), (1,)), ((), ())),
                         preferred_element_type=f32)
    hb = lax.dot_general(zg[:, H:], fcw, (((1,), (1,)), ((), ())),
                         preferred_element_type=f32)
    h = jnp.concatenate([ha, hb], axis=1) + fcb2_ref[...]
    out = ln_stats(h)
    out_ref[...] = out * flng2_ref[...] + flnb2_ref[...]


def _tc_fuse(g2, pce, fuse_w, lng2, fc_W, fcb2, flng2, flnb2):
    const = lambda shape: pl.BlockSpec(shape, lambda i: (0,) * len(shape))
    return pl.pallas_call(
        _tc_body,
        grid=(_GRID,),
        in_specs=[
            pl.BlockSpec((C, _PAIR_BLK, 128), lambda i: (0, i, 0)),
            const((C, _PAIR_BLK, 128)),
            const((1, C)),
            const((1, 128)),
            const((H, H)),
            const((1, 128)),
            const((1, 128)),
            const((1, 128)),
        ],
        out_specs=pl.BlockSpec((_PAIR_BLK, 128), lambda i: (i, 0)),
        out_shape=jax.ShapeDtypeStruct((TP, 128), jnp.float32),
    )(g2, pce, fuse_w, lng2, fc_W, fcb2, flng2, flnb2)


def kernel(x, masks, word_emb, pos_emb, chan_emb, ln_g, ln_b, fuse_w, fc_W,
           fc_b, fln_g, fln_b):
    pce = (pos_emb[None, :, :] + chan_emb[:, None, :]).reshape(C, S // 2, 128)
    pce = jnp.tile(pce, (1, _PAIR_BLK // (S // 2), 1))
    pair = lambda v: jnp.concatenate([v, v]).reshape(1, 128)
    fcb2 = pair(ln_b @ fc_W.T + fc_b)

    w2 = _tc_detranspose(word_emb.T)
    w2lin = w2.reshape(2 * VSPLIT, H)
    xi = jnp.where(x < VSPLIT, 2 * x, 2 * (x - VSPLIT) + 1)

    outs = []
    for k in range(HALF):
        xk = lax.slice_in_dim(xi, k * BH, (k + 1) * BH, axis=0)
        xt = xk.transpose(2, 0, 1).reshape(NW, NCHUNK, CHUNK)
        g = _sc_gather(xt, w2lin)
        g2 = g.reshape(C, TP, 128)
        outs.append(_tc_fuse(
            g2, pce, fuse_w.reshape(1, C), pair(ln_g), fc_W,
            fcb2, pair(fln_g), pair(fln_b),
        ))
    out = jnp.concatenate(outs, axis=0)
    return (out.reshape(B, S, H), masks)
